# baseline (speedup 1.0000x reference)
import functools

import jax
import jax.numpy as jnp
from jax import lax
from jax.experimental import pallas as pl
from jax.experimental.pallas import tpu as pltpu
from jax.experimental.pallas import tpu_sc as plsc

_LANES = 256
_GROUP = 5


def _tc_transpose_table(Wt):
    D, V = Wt.shape
    BLK = 2048

    def body(x_ref, o_ref):
        ident = jnp.eye(D, dtype=jnp.float32)
        o_ref[...] = jax.lax.dot_general(
            x_ref[...], ident, (((0,), (0,)), ((), ())),
            preferred_element_type=jnp.float32,
        )

    return pl.pallas_call(
        body,
        grid=(pl.cdiv(V, BLK),),
        in_specs=[pl.BlockSpec((D, BLK), lambda j: (0, j))],
        out_specs=pl.BlockSpec((BLK, D), lambda j: (j, 0)),
        out_shape=jax.ShapeDtypeStruct((V, D), jnp.float32),
    )(Wt)


def _tc_transpose_out(out_sm, S, B, D):
    x3 = out_sm.reshape(S, B, D)
    BLKB = 1024

    def body(x_ref, o_ref):
        ident = jnp.eye(D, dtype=jnp.float32)
        o_ref[0] = jax.lax.dot_general(
            ident, x_ref[0], (((1,), (1,)), ((), ())),
            preferred_element_type=jnp.float32,
        )

    res = pl.pallas_call(
        body,
        grid=(S, B // BLKB),
        in_specs=[pl.BlockSpec((1, BLKB, D), lambda s, j: (s, j, 0))],
        out_specs=pl.BlockSpec((1, D, BLKB), lambda s, j: (s, 0, j)),
        out_shape=jax.ShapeDtypeStruct((S, D, B), jnp.float32),
    )(x3)
    return jnp.transpose(res, (2, 0, 1))


def _sc_gather(N, D, n_idx_rows, n_groups, rows_per_w, NC):
    mesh = plsc.VectorSubcoreMesh(core_axis_name="c", subcore_axis_name="s")
    group_rows = _GROUP * _LANES

    @functools.partial(
        pl.kernel,
        mesh=mesh,
        out_type=jax.ShapeDtypeStruct((N, D), jnp.float32),
        scratch_types=[
            pltpu.VMEM((n_idx_rows, _LANES), jnp.int32),
            pltpu.VMEM((group_rows, D), jnp.float32),
            pltpu.VMEM((group_rows, D), jnp.float32),
            pltpu.SemaphoreType.DMA,
            pltpu.SemaphoreType.DMA,
            pltpu.SemaphoreType.DMA,
            pltpu.SemaphoreType.DMA,
        ],
        compiler_params=pltpu.CompilerParams(use_tc_tiling_on_sc=False),
    )
    def run(table_hbm, idx_hbm, out_hbm, idx_v, buf0, buf1, gs0, gs1, ws0, ws1):
        wid = lax.axis_index("s") * NC + lax.axis_index("c")
        idx_row0 = wid * n_idx_rows
        row0 = wid * rows_per_w
        pltpu.sync_copy(idx_hbm.at[pl.ds(idx_row0, n_idx_rows)], idx_v)

        def fire_gathers(g, buf, sem):
            for j in range(_GROUP):
                pltpu.async_copy(
                    table_hbm.at[idx_v.at[g * _GROUP + j]],
                    buf.at[pl.ds(j * _LANES, _LANES)],
                    sem,
                )

        def drain_gathers(buf, sem):
            pltpu.make_async_copy(out_hbm.at[pl.ds(0, group_rows)], buf, sem).wait()

        def fire_wb(g, buf, sem):
            pltpu.async_copy(
                buf, out_hbm.at[pl.ds(row0 + g * group_rows, group_rows)], sem
            )

        def drain_wb(buf, sem):
            pltpu.make_async_copy(buf, out_hbm.at[pl.ds(0, group_rows)], sem).wait()

        fire_gathers(0, buf0, gs0)
        fire_gathers(1, buf1, gs1)

        def outer(t, carry):
            g0 = 2 * t
            drain_gathers(buf0, gs0)
            fire_wb(g0, buf0, ws0)
            drain_gathers(buf1, gs1)
            fire_wb(g0 + 1, buf1, ws1)
            drain_wb(buf0, ws0)
            fire_gathers(g0 + 2, buf0, gs0)
            drain_wb(buf1, ws1)
            fire_gathers(g0 + 3, buf1, gs1)
            return carry

        lax.fori_loop(0, n_groups // 2 - 1, outer, 0)

        g_last = n_groups - 2
        drain_gathers(buf0, gs0)
        fire_wb(g_last, buf0, ws0)
        drain_gathers(buf1, gs1)
        fire_wb(g_last + 1, buf1, ws1)
        drain_wb(buf0, ws0)
        drain_wb(buf1, ws1)

    return run


def kernel(x, W):
    B, S = x.shape
    V, D = W.shape
    flat = x.T.reshape(-1).astype(jnp.int32)
    N = flat.shape[0]

    info = plsc.get_sparse_core_info()
    NC, NS = info.num_cores, info.num_subcores
    NW = NC * NS
    rows_per_w = N // NW
    n_idx_rows = rows_per_w // _LANES
    n_groups = n_idx_rows // _GROUP

    idx2d = flat.reshape(N // _LANES, _LANES)
    Wrm = _tc_transpose_table(W.T)
    out_sm = _sc_gather(N, D, n_idx_rows, n_groups, rows_per_w, NC)(Wrm, idx2d)
    return _tc_transpose_out(out_sm, S, B, D)


# speedup vs baseline: 1.7135x; 1.7135x over previous
import functools

import jax
import jax.numpy as jnp
from jax import lax
from jax.experimental import pallas as pl
from jax.experimental.pallas import tpu as pltpu
from jax.experimental.pallas import tpu_sc as plsc

_LANES = 256
_GROUP = 5


def _tc_transpose_table(Wt):
    D, V = Wt.shape
    BLK = 2048

    def body(x_ref, o_ref):
        ident = jnp.eye(D, dtype=jnp.float32)
        o_ref[...] = jax.lax.dot_general(
            x_ref[...], ident, (((0,), (0,)), ((), ())),
            preferred_element_type=jnp.float32,
        )

    return pl.pallas_call(
        body,
        grid=(pl.cdiv(V, BLK),),
        in_specs=[pl.BlockSpec((D, BLK), lambda j: (0, j))],
        out_specs=pl.BlockSpec((BLK, D), lambda j: (j, 0)),
        out_shape=jax.ShapeDtypeStruct((V, D), jnp.float32),
    )(Wt)


def _tc_transpose_out(out_sm, S, B, D):
    x3 = out_sm.reshape(S, B, D)
    BLKB = 1024

    def body(x_ref, o_ref):
        ident = jnp.eye(D, dtype=jnp.float32)
        o_ref[0] = jax.lax.dot_general(
            ident, x_ref[0], (((1,), (1,)), ((), ())),
            preferred_element_type=jnp.float32,
        )

    res = pl.pallas_call(
        body,
        grid=(S, B // BLKB),
        in_specs=[pl.BlockSpec((1, BLKB, D), lambda s, j: (s, j, 0))],
        out_specs=pl.BlockSpec((1, D, BLKB), lambda s, j: (s, 0, j)),
        out_shape=jax.ShapeDtypeStruct((S, D, B), jnp.float32),
    )(x3)
    return jnp.transpose(res, (2, 0, 1))


def _sc_gather(N, D, n_idx_rows, n_groups, rows_per_w, NC):
    mesh = plsc.VectorSubcoreMesh(core_axis_name="c", subcore_axis_name="s")
    group_rows = _GROUP * _LANES

    @functools.partial(
        pl.kernel,
        mesh=mesh,
        out_type=jax.ShapeDtypeStruct((N, D), jnp.float32),
        scratch_types=[
            pltpu.VMEM((n_idx_rows, _LANES), jnp.int32),
            pltpu.VMEM((group_rows, D), jnp.float32),
            pltpu.VMEM((group_rows, D), jnp.float32),
            pltpu.SemaphoreType.DMA,
            pltpu.SemaphoreType.DMA,
            pltpu.SemaphoreType.DMA,
            pltpu.SemaphoreType.DMA,
        ],
        compiler_params=pltpu.CompilerParams(use_tc_tiling_on_sc=False),
    )
    def run(table_hbm, idx_hbm, out_hbm, idx_v, buf0, buf1, gs0, gs1, ws0, ws1):
        wid = lax.axis_index("s") * NC + lax.axis_index("c")
        idx_row0 = wid * n_idx_rows
        row0 = wid * rows_per_w
        pltpu.sync_copy(idx_hbm.at[pl.ds(idx_row0, n_idx_rows)], idx_v)

        def fire_gathers(g, buf, sem):
            for j in range(_GROUP):
                pltpu.async_copy(
                    table_hbm.at[idx_v.at[g * _GROUP + j]],
                    buf.at[pl.ds(j * _LANES, _LANES)],
                    sem,
                )

        def drain_gathers(buf, sem):
            pltpu.make_async_copy(out_hbm.at[pl.ds(0, group_rows)], buf, sem).wait()

        def fire_wb(g, buf, sem):
            pltpu.async_copy(
                buf, out_hbm.at[pl.ds(row0 + g * group_rows, group_rows)], sem
            )

        def drain_wb(buf, sem):
            pltpu.make_async_copy(buf, out_hbm.at[pl.ds(0, group_rows)], sem).wait()

        fire_gathers(0, buf0, gs0)
        fire_gathers(1, buf1, gs1)

        def outer(t, carry):
            g0 = 2 * t
            drain_gathers(buf0, gs0)
            fire_wb(g0, buf0, ws0)
            drain_gathers(buf1, gs1)
            fire_wb(g0 + 1, buf1, ws1)
            drain_wb(buf0, ws0)
            fire_gathers(g0 + 2, buf0, gs0)
            drain_wb(buf1, ws1)
            fire_gathers(g0 + 3, buf1, gs1)
            return carry

        lax.fori_loop(0, n_groups // 2 - 1, outer, 0)

        g_last = n_groups - 2
        drain_gathers(buf0, gs0)
        fire_wb(g_last, buf0, ws0)
        drain_gathers(buf1, gs1)
        fire_wb(g_last + 1, buf1, ws1)
        drain_wb(buf0, ws0)
        drain_wb(buf1, ws1)

    return run


def kernel(x, W):
    B, S = x.shape
    V, D = W.shape
    flat = x.reshape(-1).astype(jnp.int32)
    N = flat.shape[0]

    info = plsc.get_sparse_core_info()
    NC, NS = info.num_cores, info.num_subcores
    NW = NC * NS
    rows_per_w = N // NW
    n_idx_rows = rows_per_w // _LANES
    n_groups = n_idx_rows // _GROUP

    idx2d = flat.reshape(N // _LANES, _LANES)
    out = _sc_gather(N, D, n_idx_rows, n_groups, rows_per_w, NC)(W, idx2d)
    return out.reshape(B, S, D)
